# baseline (speedup 1.0000x reference)
import functools

import jax
import jax.numpy as jnp
from jax import lax
from jax.experimental import pallas as pl
from jax.experimental.pallas import tpu as pltpu
from jax.experimental.pallas import tpu_sc as plsc

E = 64
TOPK = 2
D = 1024
DFF = 1024
T = 2048
TT = T * TOPK

B = 128
NB = TT // B + E
NP = NB * B
DBLK = 512
NJ = DFF // DBLK

_TB = 256

_NW = 32
_CH = 16
_ROUNDS = NP // _CH // _NW
_TW = T // _NW

@functools.cache
def _sc_mesh():
    return plsc.VectorSubcoreMesh(core_axis_name="c", subcore_axis_name="s")


def _router_body(x_ref, wg_ref, topi_ref, topw_ref):
    x = x_ref[...]
    logits = jnp.dot(x, wg_ref[...], preferred_element_type=jnp.float32)
    lane = lax.broadcasted_iota(jnp.int32, logits.shape, 1)
    m1 = jnp.max(logits, axis=1, keepdims=True)
    i1 = jnp.min(jnp.where(logits == m1, lane, E), axis=1, keepdims=True)
    masked = jnp.where(lane == i1, -jnp.inf, logits)
    m2 = jnp.max(masked, axis=1, keepdims=True)
    i2 = jnp.min(jnp.where(masked == m2, lane, E), axis=1, keepdims=True)
    w1 = jax.nn.sigmoid(m1 - m2)
    topi_ref[...] = jnp.concatenate([i1, i2], axis=1).astype(jnp.int32)
    topw_ref[...] = jnp.concatenate([w1, 1.0 - w1], axis=1)


_router = pl.pallas_call(
    _router_body,
    grid=(T // _TB,),
    in_specs=[
        pl.BlockSpec((_TB, D), lambda t: (t, 0)),
        pl.BlockSpec((D, E), lambda t: (0, 0)),
    ],
    out_specs=[
        pl.BlockSpec((_TB, TOPK), lambda t: (t, 0)),
        pl.BlockSpec((_TB, TOPK), lambda t: (t, 0)),
    ],
    out_shape=[
        jax.ShapeDtypeStruct((T, TOPK), jnp.int32),
        jax.ShapeDtypeStruct((T, TOPK), jnp.float32),
    ],
)


def _dispatch(topi, topw):
    eflat = topi.reshape(-1)
    onehot = (eflat[:, None] == jnp.arange(E, dtype=jnp.int32)[None, :]).astype(jnp.int32)
    pos = jnp.cumsum(onehot, axis=0)
    counts = pos[-1]
    rank = jnp.sum(onehot * pos, axis=1) - 1
    nblk = (counts + B - 1) // B
    blk_cum = jnp.cumsum(nblk)
    nb_active = blk_cum[-1]
    bb = jnp.arange(NB, dtype=jnp.int32)
    blk_expert = jnp.sum((bb[:, None] >= blk_cum[None, :]).astype(jnp.int32), axis=1)
    blk_expert = jnp.minimum(blk_expert, E - 1)
    sp = jnp.concatenate([blk_expert, nb_active[None]]).astype(jnp.int32)
    pad_off = (blk_cum - nblk) * B
    ps = pad_off[eflat] + rank
    padded_token = jnp.zeros((NP,), jnp.int32).at[ps].set(
        jnp.arange(TT, dtype=jnp.int32) // TOPK)
    w_pad = jnp.zeros((NP,), jnp.float32).at[ps].set(topw.reshape(-1))
    cpos = ps.reshape(T, TOPK)
    return sp, padded_token, w_pad, cpos


@functools.cache
def _sc_gather():
    @functools.partial(
        pl.kernel,
        out_type=jax.ShapeDtypeStruct((NP, D), jnp.float32),
        mesh=_sc_mesh(),
        scratch_types=[
            pltpu.VMEM((_ROUNDS, _CH), jnp.int32),
            pltpu.VMEM((4, _CH, D), jnp.float32),
            pltpu.SemaphoreType.DMA,
            pltpu.SemaphoreType.DMA,
            pltpu.SemaphoreType.DMA,
            pltpu.SemaphoreType.DMA,
            pltpu.SemaphoreType.DMA,
            pltpu.SemaphoreType.DMA,
            pltpu.SemaphoreType.DMA,
            pltpu.SemaphoreType.DMA,
        ],
    )
    def gather(x_hbm, tok_hbm, xs_hbm, idx_v, rows_v,
               g0, g1, g2, g3, s0, s1, s2, s3):
        wid = lax.axis_index("s") * 2 + lax.axis_index("c")
        gsem = (g0, g1, g2, g3)
        ssem = (s0, s1, s2, s3)
        gcp = [None] * _ROUNDS
        scp = [None] * _ROUNDS

        def base(r):
            return (r * _NW + wid) * _CH

        def start_g(r):
            pltpu.sync_copy(tok_hbm.at[pl.ds(base(r), _CH)], idx_v.at[r])
            gcp[r] = pltpu.async_copy(
                x_hbm.at[idx_v.at[r]], rows_v.at[r % 4], gsem[r % 4])

        def start_s(r):
            scp[r] = pltpu.async_copy(
                rows_v.at[r % 4], xs_hbm.at[pl.ds(base(r), _CH)], ssem[r % 4])

        for r in range(_ROUNDS):
            if r >= 4:
                scp[r - 4].wait()
            start_g(r)
            if r >= 2:
                gcp[r - 2].wait()
                start_s(r - 2)
        for r in (_ROUNDS - 2, _ROUNDS - 1):
            gcp[r].wait()
            start_s(r)
        for r in (_ROUNDS - 4, _ROUNDS - 3, _ROUNDS - 2, _ROUNDS - 1):
            scp[r].wait()

    return gather


def _ffn_body(sp_ref, xs_ref, w_ref, wg_ref, wu_ref, wd_ref, ys_ref):
    b = pl.program_id(0)
    j = pl.program_id(1)
    nact = sp_ref[NB]

    @pl.when(b < nact)
    def _():
        x = xs_ref[...]
        g = jnp.dot(x, wg_ref[0], preferred_element_type=jnp.float32)
        u = jnp.dot(x, wu_ref[0], preferred_element_type=jnp.float32)
        h = g * jax.nn.sigmoid(g) * u
        part = jnp.dot(h, wd_ref[0], preferred_element_type=jnp.float32) * w_ref[...]

        @pl.when(j == 0)
        def _():
            ys_ref[...] = part

        @pl.when(j > 0)
        def _():
            ys_ref[...] += part


_ffn = pl.pallas_call(
    _ffn_body,
    grid_spec=pltpu.PrefetchScalarGridSpec(
        num_scalar_prefetch=1,
        grid=(NB, NJ),
        in_specs=[
            pl.BlockSpec((B, D), lambda b, j, sp: (b, 0)),
            pl.BlockSpec((B, 1), lambda b, j, sp: (b, 0)),
            pl.BlockSpec((1, D, DBLK), lambda b, j, sp: (sp[b], 0, j)),
            pl.BlockSpec((1, D, DBLK), lambda b, j, sp: (sp[b], 0, j)),
            pl.BlockSpec((1, DBLK, D), lambda b, j, sp: (sp[b], j, 0)),
        ],
        out_specs=pl.BlockSpec((B, D), lambda b, j, sp: (b, 0)),
    ),
    out_shape=jax.ShapeDtypeStruct((NP, D), jnp.float32),
    compiler_params=pltpu.CompilerParams(
        dimension_semantics=("arbitrary", "arbitrary")),
)


@functools.cache
def _sc_combine():
    @functools.partial(
        pl.kernel,
        out_type=jax.ShapeDtypeStruct((T, D), jnp.float32),
        mesh=_sc_mesh(),
        scratch_types=[
            pltpu.VMEM((16,), jnp.int32),
            pltpu.VMEM((16,), jnp.int32),
            pltpu.VMEM((16, D), jnp.float32),
            pltpu.VMEM((16, D), jnp.float32),
            pltpu.SemaphoreType.DMA,
            pltpu.SemaphoreType.DMA,
        ],
    )
    def combine(ys_hbm, c0_hbm, c1_hbm, out_hbm,
                i0_v, i1_v, a_v, b_v, sem0, sem1):
        wid = lax.axis_index("s") * 2 + lax.axis_index("c")
        for r in range(_TW // 16):
            base = wid * _TW + r * 16
            pltpu.sync_copy(c0_hbm.at[pl.ds(base, 16)], i0_v)
            pltpu.sync_copy(c1_hbm.at[pl.ds(base, 16)], i1_v)
            cp0 = pltpu.async_copy(ys_hbm.at[i0_v], a_v, sem0)
            cp1 = pltpu.async_copy(ys_hbm.at[i1_v], b_v, sem1)
            cp0.wait()
            cp1.wait()

            def body(i, carry):
                l = i // (D // 16)
                sl = pl.ds((i % (D // 16)) * 16, 16)
                a_v[l, sl] = a_v[l, sl] + b_v[l, sl]
                return carry

            lax.fori_loop(0, 16 * (D // 16), body, 0, unroll=4)
            pltpu.sync_copy(a_v, out_hbm.at[pl.ds(base, 16)])

    return combine


def kernel(hidden_states, Wg, W_gate, W_up, W_down):
    x = hidden_states
    topi, topw = _router(x, Wg)
    sp, padded_token, w_pad, cpos = _dispatch(topi, topw)
    xs = _sc_gather()(x, padded_token)
    ys = _ffn(sp, xs, w_pad[:, None], W_gate, W_up, W_down)
    return _sc_combine()(ys, cpos[:, 0], cpos[:, 1])


# speedup vs baseline: 1.6167x; 1.6167x over previous
import functools

import jax
import jax.numpy as jnp
from jax import lax
from jax.experimental import pallas as pl
from jax.experimental.pallas import tpu as pltpu
from jax.experimental.pallas import tpu_sc as plsc

E = 64
TOPK = 2
D = 1024
DFF = 1024
T = 2048
TT = T * TOPK

B = 128
NB = TT // B + E
NP = NB * B
DBLK = 512
NJ = DFF // DBLK

_TB = 256

_NW = 32
_CH = 16
_ROUNDS = NP // _CH // _NW
_TW = T // _NW

@functools.cache
def _sc_mesh():
    return plsc.VectorSubcoreMesh(core_axis_name="c", subcore_axis_name="s")


def _router_body(x_ref, wg_ref, topi_ref, topw_ref):
    x = x_ref[...]
    logits = jnp.dot(x, wg_ref[...], preferred_element_type=jnp.float32)
    lane = lax.broadcasted_iota(jnp.int32, logits.shape, 1)
    m1 = jnp.max(logits, axis=1, keepdims=True)
    i1 = jnp.min(jnp.where(logits == m1, lane, E), axis=1, keepdims=True)
    masked = jnp.where(lane == i1, -jnp.inf, logits)
    m2 = jnp.max(masked, axis=1, keepdims=True)
    i2 = jnp.min(jnp.where(masked == m2, lane, E), axis=1, keepdims=True)
    w1 = jax.nn.sigmoid(m1 - m2)
    topi_ref[...] = jnp.concatenate([i1, i2], axis=1).astype(jnp.int32)
    topw_ref[...] = jnp.concatenate([w1, 1.0 - w1], axis=1)


_router = pl.pallas_call(
    _router_body,
    grid=(T // _TB,),
    in_specs=[
        pl.BlockSpec((_TB, D), lambda t: (t, 0)),
        pl.BlockSpec((D, E), lambda t: (0, 0)),
    ],
    out_specs=[
        pl.BlockSpec((_TB, TOPK), lambda t: (t, 0)),
        pl.BlockSpec((_TB, TOPK), lambda t: (t, 0)),
    ],
    out_shape=[
        jax.ShapeDtypeStruct((T, TOPK), jnp.int32),
        jax.ShapeDtypeStruct((T, TOPK), jnp.float32),
    ],
)


def _dispatch(topi, topw):
    eflat = topi.reshape(-1)
    onehot = (eflat[:, None] == jnp.arange(E, dtype=jnp.int32)[None, :]).astype(jnp.int32)
    pos = jnp.cumsum(onehot, axis=0)
    counts = pos[-1]
    rank = jnp.sum(onehot * pos, axis=1) - 1
    nblk = (counts + B - 1) // B
    blk_cum = jnp.cumsum(nblk)
    nb_active = blk_cum[-1]
    bb = jnp.arange(NB, dtype=jnp.int32)
    blk_expert = jnp.sum((bb[:, None] >= blk_cum[None, :]).astype(jnp.int32), axis=1)
    blk_expert = jnp.minimum(blk_expert, E - 1)
    sp = jnp.concatenate([blk_expert, nb_active[None]]).astype(jnp.int32)
    pad_off = (blk_cum - nblk) * B
    ps = pad_off[eflat] + rank
    padded_token = (jnp.arange(NP, dtype=jnp.int32) % T).at[ps].set(
        jnp.arange(TT, dtype=jnp.int32) // TOPK)
    w_pad = jnp.zeros((NP,), jnp.float32).at[ps].set(topw.reshape(-1))
    cpos = ps.reshape(T, TOPK)
    return sp, padded_token, w_pad, cpos


@functools.cache
def _sc_gather():
    @functools.partial(
        pl.kernel,
        out_type=jax.ShapeDtypeStruct((NP, D), jnp.float32),
        mesh=_sc_mesh(),
        scratch_types=[
            pltpu.VMEM((_ROUNDS, _CH), jnp.int32),
            pltpu.VMEM((4, _CH, D), jnp.float32),
            pltpu.SemaphoreType.DMA,
            pltpu.SemaphoreType.DMA,
            pltpu.SemaphoreType.DMA,
            pltpu.SemaphoreType.DMA,
            pltpu.SemaphoreType.DMA,
            pltpu.SemaphoreType.DMA,
            pltpu.SemaphoreType.DMA,
            pltpu.SemaphoreType.DMA,
        ],
    )
    def gather(x_hbm, tok_hbm, xs_hbm, idx_v, rows_v,
               g0, g1, g2, g3, s0, s1, s2, s3):
        wid = lax.axis_index("s") * 2 + lax.axis_index("c")
        gsem = (g0, g1, g2, g3)
        ssem = (s0, s1, s2, s3)
        gcp = [None] * _ROUNDS
        scp = [None] * _ROUNDS

        def base(r):
            return (r * _NW + wid) * _CH

        def start_g(r):
            pltpu.sync_copy(tok_hbm.at[pl.ds(base(r), _CH)], idx_v.at[r])
            gcp[r] = pltpu.async_copy(
                x_hbm.at[idx_v.at[r]], rows_v.at[r % 4], gsem[r % 4])

        def start_s(r):
            scp[r] = pltpu.async_copy(
                rows_v.at[r % 4], xs_hbm.at[pl.ds(base(r), _CH)], ssem[r % 4])

        for r in range(_ROUNDS):
            if r >= 4:
                scp[r - 4].wait()
            start_g(r)
            if r >= 2:
                gcp[r - 2].wait()
                start_s(r - 2)
        for r in (_ROUNDS - 2, _ROUNDS - 1):
            gcp[r].wait()
            start_s(r)
        for r in (_ROUNDS - 4, _ROUNDS - 3, _ROUNDS - 2, _ROUNDS - 1):
            scp[r].wait()

    return gather


def _ffn_body(sp_ref, xs_ref, w_ref, wg_ref, wu_ref, wd_ref, ys_ref):
    b = pl.program_id(0)
    j = pl.program_id(1)
    nact = sp_ref[NB]

    @pl.when(b < nact)
    def _():
        x = xs_ref[...]
        g = jnp.dot(x, wg_ref[0], preferred_element_type=jnp.float32)
        u = jnp.dot(x, wu_ref[0], preferred_element_type=jnp.float32)
        h = g * jax.nn.sigmoid(g) * u
        part = jnp.dot(h, wd_ref[0], preferred_element_type=jnp.float32) * w_ref[...]

        @pl.when(j == 0)
        def _():
            ys_ref[...] = part

        @pl.when(j > 0)
        def _():
            ys_ref[...] += part


_ffn = pl.pallas_call(
    _ffn_body,
    grid_spec=pltpu.PrefetchScalarGridSpec(
        num_scalar_prefetch=1,
        grid=(NB, NJ),
        in_specs=[
            pl.BlockSpec((B, D), lambda b, j, sp: (b, 0)),
            pl.BlockSpec((B, 1), lambda b, j, sp: (b, 0)),
            pl.BlockSpec((1, D, DBLK), lambda b, j, sp: (sp[b], 0, j)),
            pl.BlockSpec((1, D, DBLK), lambda b, j, sp: (sp[b], 0, j)),
            pl.BlockSpec((1, DBLK, D), lambda b, j, sp: (sp[b], j, 0)),
        ],
        out_specs=pl.BlockSpec((B, D), lambda b, j, sp: (b, 0)),
    ),
    out_shape=jax.ShapeDtypeStruct((NP, D), jnp.float32),
    compiler_params=pltpu.CompilerParams(
        dimension_semantics=("arbitrary", "arbitrary")),
)


@functools.cache
def _sc_combine():
    @functools.partial(
        pl.kernel,
        out_type=jax.ShapeDtypeStruct((T, D), jnp.float32),
        mesh=_sc_mesh(),
        scratch_types=[
            pltpu.VMEM((16,), jnp.int32),
            pltpu.VMEM((16,), jnp.int32),
            pltpu.VMEM((16, D), jnp.float32),
            pltpu.VMEM((16, D), jnp.float32),
            pltpu.SemaphoreType.DMA,
            pltpu.SemaphoreType.DMA,
        ],
    )
    def combine(ys_hbm, c0_hbm, c1_hbm, out_hbm,
                i0_v, i1_v, a_v, b_v, sem0, sem1):
        wid = lax.axis_index("s") * 2 + lax.axis_index("c")
        for r in range(_TW // 16):
            base = wid * _TW + r * 16
            pltpu.sync_copy(c0_hbm.at[pl.ds(base, 16)], i0_v)
            pltpu.sync_copy(c1_hbm.at[pl.ds(base, 16)], i1_v)
            cp0 = pltpu.async_copy(ys_hbm.at[i0_v], a_v, sem0)
            cp1 = pltpu.async_copy(ys_hbm.at[i1_v], b_v, sem1)
            cp0.wait()
            cp1.wait()

            def body(i, carry):
                l = i // (D // 16)
                sl = pl.ds((i % (D // 16)) * 16, 16)
                a_v[l, sl] = a_v[l, sl] + b_v[l, sl]
                return carry

            lax.fori_loop(0, 16 * (D // 16), body, 0, unroll=4)
            pltpu.sync_copy(a_v, out_hbm.at[pl.ds(base, 16)])

    return combine


def kernel(hidden_states, Wg, W_gate, W_up, W_down):
    x = hidden_states
    topi, topw = _router(x, Wg)
    sp, padded_token, w_pad, cpos = _dispatch(topi, topw)
    xs = _sc_gather()(x, padded_token)
    ys = _ffn(sp, xs, w_pad[:, None], W_gate, W_up, W_down)
    return _sc_combine()(ys, cpos[:, 0], cpos[:, 1])


# speedup vs baseline: 1.7073x; 1.0560x over previous
import functools

import jax
import jax.numpy as jnp
from jax import lax
from jax.experimental import pallas as pl
from jax.experimental.pallas import tpu as pltpu
from jax.experimental.pallas import tpu_sc as plsc

E = 64
TOPK = 2
D = 1024
DFF = 1024
T = 2048
TT = T * TOPK

B = 128
NB = TT // B + E
NP = NB * B
DBLK = 512
NJ = DFF // DBLK

_TB = 256

_NW = 32
_CH = 16
_ROUNDS = NP // _CH // _NW
_TW = T // _NW

@functools.cache
def _sc_mesh():
    return plsc.VectorSubcoreMesh(core_axis_name="c", subcore_axis_name="s")


def _router_body(x_ref, wg_ref, topi_ref, topw_ref):
    x = x_ref[...]
    logits = jnp.dot(x, wg_ref[...], preferred_element_type=jnp.float32)
    lane = lax.broadcasted_iota(jnp.int32, logits.shape, 1)
    m1 = jnp.max(logits, axis=1, keepdims=True)
    i1 = jnp.min(jnp.where(logits == m1, lane, E), axis=1, keepdims=True)
    masked = jnp.where(lane == i1, -jnp.inf, logits)
    m2 = jnp.max(masked, axis=1, keepdims=True)
    i2 = jnp.min(jnp.where(masked == m2, lane, E), axis=1, keepdims=True)
    w1 = jax.nn.sigmoid(m1 - m2)
    topi_ref[...] = jnp.concatenate([i1, i2], axis=1).astype(jnp.int32)
    topw_ref[...] = jnp.concatenate([w1, 1.0 - w1], axis=1)


_router = pl.pallas_call(
    _router_body,
    grid=(T // _TB,),
    in_specs=[
        pl.BlockSpec((_TB, D), lambda t: (t, 0)),
        pl.BlockSpec((D, E), lambda t: (0, 0)),
    ],
    out_specs=[
        pl.BlockSpec((_TB, TOPK), lambda t: (t, 0)),
        pl.BlockSpec((_TB, TOPK), lambda t: (t, 0)),
    ],
    out_shape=[
        jax.ShapeDtypeStruct((T, TOPK), jnp.int32),
        jax.ShapeDtypeStruct((T, TOPK), jnp.float32),
    ],
)


def _dispatch(topi, topw):
    eflat = topi.reshape(-1)
    onehot = (eflat[:, None] == jnp.arange(E, dtype=jnp.int32)[None, :]).astype(jnp.int32)
    pos = jnp.cumsum(onehot, axis=0)
    counts = pos[-1]
    rank = jnp.sum(onehot * pos, axis=1) - 1
    nblk = (counts + B - 1) // B
    blk_cum = jnp.cumsum(nblk)
    nb_active = blk_cum[-1]
    bb = jnp.arange(NB, dtype=jnp.int32)
    blk_expert = jnp.sum((bb[:, None] >= blk_cum[None, :]).astype(jnp.int32), axis=1)
    blk_expert = jnp.minimum(blk_expert, E - 1)
    sp = jnp.concatenate([blk_expert, nb_active[None]]).astype(jnp.int32)
    pad_off = (blk_cum - nblk) * B
    ps = pad_off[eflat] + rank
    w_pad = jnp.zeros((NP,), jnp.float32).at[ps].set(topw.reshape(-1))
    cpos = ps.reshape(T, TOPK)
    ps2 = ps.reshape(_NW, T // _NW, TOPK).transpose(0, 2, 1)
    return sp, ps2, w_pad, cpos


@functools.cache
def _sc_dispatch():
    @functools.partial(
        pl.kernel,
        out_type=jax.ShapeDtypeStruct((NP, D), jnp.float32),
        mesh=_sc_mesh(),
        scratch_types=[
            pltpu.VMEM((TOPK, T // _NW), jnp.int32),
            pltpu.VMEM((T // _NW, D), jnp.float32),
            pltpu.SemaphoreType.DMA,
            pltpu.SemaphoreType.DMA,
        ],
    )
    def dispatch(x_hbm, ps2_hbm, xs_hbm, idx_v, rows_v, s0, s1):
        wid = lax.axis_index("s") * 2 + lax.axis_index("c")
        tw = T // _NW
        pltpu.sync_copy(x_hbm.at[pl.ds(wid * tw, tw)], rows_v)
        pltpu.sync_copy(ps2_hbm.at[wid], idx_v)
        cp0 = pltpu.async_copy(rows_v, xs_hbm.at[idx_v.at[0]], s0)
        cp1 = pltpu.async_copy(rows_v, xs_hbm.at[idx_v.at[1]], s1)
        cp0.wait()
        cp1.wait()

    return dispatch


def _ffn_body(sp_ref, xs_ref, w_ref, wg_ref, wu_ref, wd_ref, ys_ref):
    b = pl.program_id(0)
    j = pl.program_id(1)
    nact = sp_ref[NB]

    @pl.when(b < nact)
    def _():
        x = xs_ref[...]
        g = jnp.dot(x, wg_ref[0], preferred_element_type=jnp.float32)
        u = jnp.dot(x, wu_ref[0], preferred_element_type=jnp.float32)
        h = g * jax.nn.sigmoid(g) * u
        part = jnp.dot(h, wd_ref[0], preferred_element_type=jnp.float32) * w_ref[...]

        @pl.when(j == 0)
        def _():
            ys_ref[...] = part

        @pl.when(j > 0)
        def _():
            ys_ref[...] += part


_ffn = pl.pallas_call(
    _ffn_body,
    grid_spec=pltpu.PrefetchScalarGridSpec(
        num_scalar_prefetch=1,
        grid=(NB, NJ),
        in_specs=[
            pl.BlockSpec((B, D), lambda b, j, sp: (b, 0)),
            pl.BlockSpec((B, 1), lambda b, j, sp: (b, 0)),
            pl.BlockSpec((1, D, DBLK), lambda b, j, sp: (sp[b], 0, j)),
            pl.BlockSpec((1, D, DBLK), lambda b, j, sp: (sp[b], 0, j)),
            pl.BlockSpec((1, DBLK, D), lambda b, j, sp: (sp[b], j, 0)),
        ],
        out_specs=pl.BlockSpec((B, D), lambda b, j, sp: (b, 0)),
    ),
    out_shape=jax.ShapeDtypeStruct((NP, D), jnp.float32),
    compiler_params=pltpu.CompilerParams(
        dimension_semantics=("arbitrary", "arbitrary")),
)


@functools.cache
def _sc_combine():
    @functools.partial(
        pl.kernel,
        out_type=jax.ShapeDtypeStruct((T, D), jnp.float32),
        mesh=_sc_mesh(),
        scratch_types=[
            pltpu.VMEM((16,), jnp.int32),
            pltpu.VMEM((16,), jnp.int32),
            pltpu.VMEM((16, D), jnp.float32),
            pltpu.VMEM((16, D), jnp.float32),
            pltpu.SemaphoreType.DMA,
            pltpu.SemaphoreType.DMA,
        ],
    )
    def combine(ys_hbm, c0_hbm, c1_hbm, out_hbm,
                i0_v, i1_v, a_v, b_v, sem0, sem1):
        wid = lax.axis_index("s") * 2 + lax.axis_index("c")
        for r in range(_TW // 16):
            base = wid * _TW + r * 16
            pltpu.sync_copy(c0_hbm.at[pl.ds(base, 16)], i0_v)
            pltpu.sync_copy(c1_hbm.at[pl.ds(base, 16)], i1_v)
            cp0 = pltpu.async_copy(ys_hbm.at[i0_v], a_v, sem0)
            cp1 = pltpu.async_copy(ys_hbm.at[i1_v], b_v, sem1)
            cp0.wait()
            cp1.wait()

            def body(i, carry):
                l = i // (D // 16)
                sl = pl.ds((i % (D // 16)) * 16, 16)
                a_v[l, sl] = a_v[l, sl] + b_v[l, sl]
                return carry

            lax.fori_loop(0, 16 * (D // 16), body, 0, unroll=4)
            pltpu.sync_copy(a_v, out_hbm.at[pl.ds(base, 16)])

    return combine


def kernel(hidden_states, Wg, W_gate, W_up, W_down):
    x = hidden_states
    topi, topw = _router(x, Wg)
    sp, ps2, w_pad, cpos = _dispatch(topi, topw)
    xs = _sc_dispatch()(x, ps2)
    ys = _ffn(sp, xs, w_pad[:, None], W_gate, W_up, W_down)
    return _sc_combine()(ys, cpos[:, 0], cpos[:, 1])


# speedup vs baseline: 1.7370x; 1.0174x over previous
import functools

import jax
import jax.numpy as jnp
from jax import lax
from jax.experimental import pallas as pl
from jax.experimental.pallas import tpu as pltpu
from jax.experimental.pallas import tpu_sc as plsc

E = 64
TOPK = 2
D = 1024
DFF = 1024
T = 2048
TT = T * TOPK

B = 128
NB = TT // B + E
NP = NB * B
DBLK = 512
NJ = DFF // DBLK

_TB = 256

_NW = 32
_CH = 16
_ROUNDS = NP // _CH // _NW
_TW = T // _NW

@functools.cache
def _sc_mesh():
    return plsc.VectorSubcoreMesh(core_axis_name="c", subcore_axis_name="s")


def _router_body(x_ref, wg_ref, topi_ref, topw_ref):
    x = x_ref[...]
    logits = jnp.dot(x, wg_ref[...], preferred_element_type=jnp.float32)
    lane = lax.broadcasted_iota(jnp.int32, logits.shape, 1)
    m1 = jnp.max(logits, axis=1, keepdims=True)
    i1 = jnp.min(jnp.where(logits == m1, lane, E), axis=1, keepdims=True)
    masked = jnp.where(lane == i1, -jnp.inf, logits)
    m2 = jnp.max(masked, axis=1, keepdims=True)
    i2 = jnp.min(jnp.where(masked == m2, lane, E), axis=1, keepdims=True)
    w1 = jax.nn.sigmoid(m1 - m2)
    topi_ref[...] = jnp.concatenate([i1, i2], axis=1).astype(jnp.int32)
    topw_ref[...] = jnp.concatenate([w1, 1.0 - w1], axis=1)


_router = pl.pallas_call(
    _router_body,
    grid=(T // _TB,),
    in_specs=[
        pl.BlockSpec((_TB, D), lambda t: (t, 0)),
        pl.BlockSpec((D, E), lambda t: (0, 0)),
    ],
    out_specs=[
        pl.BlockSpec((_TB, TOPK), lambda t: (t, 0)),
        pl.BlockSpec((_TB, TOPK), lambda t: (t, 0)),
    ],
    out_shape=[
        jax.ShapeDtypeStruct((T, TOPK), jnp.int32),
        jax.ShapeDtypeStruct((T, TOPK), jnp.float32),
    ],
)


def _dispatch(topi, topw):
    eflat = topi.reshape(-1)
    onehot = (eflat[:, None] == jnp.arange(E, dtype=jnp.int32)[None, :]).astype(jnp.int32)
    pos = jnp.cumsum(onehot, axis=0)
    counts = pos[-1]
    rank = jnp.sum(onehot * pos, axis=1) - 1
    nblk = (counts + B - 1) // B
    blk_cum = jnp.cumsum(nblk)
    nb_active = blk_cum[-1]
    bb = jnp.arange(NB, dtype=jnp.int32)
    blk_expert = jnp.sum((bb[:, None] >= blk_cum[None, :]).astype(jnp.int32), axis=1)
    blk_expert = jnp.minimum(blk_expert, E - 1)
    sp = jnp.concatenate([blk_expert, nb_active[None]]).astype(jnp.int32)
    pad_off = (blk_cum - nblk) * B
    ps = pad_off[eflat] + rank
    w_pad = jnp.zeros((NP,), jnp.float32).at[ps].set(topw.reshape(-1))
    cpos = ps.reshape(T, TOPK)
    ps2 = ps.reshape(_NW, T // _NW, TOPK).transpose(0, 2, 1)
    return sp, ps2, w_pad, cpos


@functools.cache
def _sc_dispatch():
    @functools.partial(
        pl.kernel,
        out_type=jax.ShapeDtypeStruct((NP, D), jnp.float32),
        mesh=_sc_mesh(),
        scratch_types=[
            pltpu.VMEM((TOPK, T // _NW), jnp.int32),
            pltpu.VMEM((T // _NW, D), jnp.float32),
            pltpu.SemaphoreType.DMA,
            pltpu.SemaphoreType.DMA,
        ],
    )
    def dispatch(x_hbm, ps2_hbm, xs_hbm, idx_v, rows_v, s0, s1):
        wid = lax.axis_index("s") * 2 + lax.axis_index("c")
        tw = T // _NW
        pltpu.sync_copy(x_hbm.at[pl.ds(wid * tw, tw)], rows_v)
        pltpu.sync_copy(ps2_hbm.at[wid], idx_v)
        cp0 = pltpu.async_copy(rows_v, xs_hbm.at[idx_v.at[0]], s0)
        cp1 = pltpu.async_copy(rows_v, xs_hbm.at[idx_v.at[1]], s1)
        cp0.wait()
        cp1.wait()

    return dispatch


def _ffn_body(sp_ref, xs_ref, w_ref, wg_ref, wu_ref, wd_ref, ys_ref):
    b = pl.program_id(0)
    j = pl.program_id(1)
    nact = sp_ref[NB]

    @pl.when(b < nact)
    def _():
        x = xs_ref[...]
        g = jnp.dot(x, wg_ref[0], preferred_element_type=jnp.float32)
        u = jnp.dot(x, wu_ref[0], preferred_element_type=jnp.float32)
        h = g * jax.nn.sigmoid(g) * u
        part = jnp.dot(h, wd_ref[0], preferred_element_type=jnp.float32) * w_ref[...]

        @pl.when(j == 0)
        def _():
            ys_ref[...] = part

        @pl.when(j > 0)
        def _():
            ys_ref[...] += part


_ffn = pl.pallas_call(
    _ffn_body,
    grid_spec=pltpu.PrefetchScalarGridSpec(
        num_scalar_prefetch=1,
        grid=(NB, NJ),
        in_specs=[
            pl.BlockSpec(
                (B, D), lambda b, j, sp: (jnp.minimum(b, sp[NB] - 1), 0)),
            pl.BlockSpec(
                (B, 1), lambda b, j, sp: (jnp.minimum(b, sp[NB] - 1), 0)),
            pl.BlockSpec(
                (1, D, DBLK),
                lambda b, j, sp: (sp[jnp.minimum(b, sp[NB] - 1)], 0, j)),
            pl.BlockSpec(
                (1, D, DBLK),
                lambda b, j, sp: (sp[jnp.minimum(b, sp[NB] - 1)], 0, j)),
            pl.BlockSpec(
                (1, DBLK, D),
                lambda b, j, sp: (sp[jnp.minimum(b, sp[NB] - 1)], j, 0)),
        ],
        out_specs=pl.BlockSpec(
            (B, D), lambda b, j, sp: (jnp.minimum(b, sp[NB] - 1), 0)),
    ),
    out_shape=jax.ShapeDtypeStruct((NP, D), jnp.float32),
    compiler_params=pltpu.CompilerParams(
        dimension_semantics=("arbitrary", "arbitrary")),
)


@functools.cache
def _sc_combine():
    @functools.partial(
        pl.kernel,
        out_type=jax.ShapeDtypeStruct((T, D), jnp.float32),
        mesh=_sc_mesh(),
        scratch_types=[
            pltpu.VMEM((16,), jnp.int32),
            pltpu.VMEM((16,), jnp.int32),
            pltpu.VMEM((16, D), jnp.float32),
            pltpu.VMEM((16, D), jnp.float32),
            pltpu.SemaphoreType.DMA,
            pltpu.SemaphoreType.DMA,
        ],
    )
    def combine(ys_hbm, c0_hbm, c1_hbm, out_hbm,
                i0_v, i1_v, a_v, b_v, sem0, sem1):
        wid = lax.axis_index("s") * 2 + lax.axis_index("c")
        for r in range(_TW // 16):
            base = wid * _TW + r * 16
            pltpu.sync_copy(c0_hbm.at[pl.ds(base, 16)], i0_v)
            pltpu.sync_copy(c1_hbm.at[pl.ds(base, 16)], i1_v)
            cp0 = pltpu.async_copy(ys_hbm.at[i0_v], a_v, sem0)
            cp1 = pltpu.async_copy(ys_hbm.at[i1_v], b_v, sem1)
            cp0.wait()
            cp1.wait()

            def body(i, carry):
                l = i // (D // 16)
                sl = pl.ds((i % (D // 16)) * 16, 16)
                a_v[l, sl] = a_v[l, sl] + b_v[l, sl]
                return carry

            lax.fori_loop(0, 16 * (D // 16), body, 0, unroll=4)
            pltpu.sync_copy(a_v, out_hbm.at[pl.ds(base, 16)])

    return combine


def kernel(hidden_states, Wg, W_gate, W_up, W_down):
    x = hidden_states
    topi, topw = _router(x, Wg)
    sp, ps2, w_pad, cpos = _dispatch(topi, topw)
    xs = _sc_dispatch()(x, ps2)
    ys = _ffn(sp, xs, w_pad[:, None], W_gate, W_up, W_down)
    return _sc_combine()(ys, cpos[:, 0], cpos[:, 1])


# speedup vs baseline: 2.3410x; 1.3478x over previous
import functools

import jax
import jax.numpy as jnp
from jax import lax
from jax.experimental import pallas as pl
from jax.experimental.pallas import tpu as pltpu
from jax.experimental.pallas import tpu_sc as plsc

E = 64
TOPK = 2
D = 1024
DFF = 1024
T = 2048
TT = T * TOPK

B = 128
NB = TT // B + E
NP = NB * B
DBLK = 1024
NJ = DFF // DBLK

_TB = 256

_NW = 32
_CH = 16
_ROUNDS = NP // _CH // _NW
_TW = T // _NW

@functools.cache
def _sc_mesh():
    return plsc.VectorSubcoreMesh(core_axis_name="c", subcore_axis_name="s")


def _router_body(x_ref, wg_ref, topi_ref, topw_ref):
    x = x_ref[...]
    logits = jnp.dot(x, wg_ref[...], preferred_element_type=jnp.float32)
    lane = lax.broadcasted_iota(jnp.int32, logits.shape, 1)
    m1 = jnp.max(logits, axis=1, keepdims=True)
    i1 = jnp.min(jnp.where(logits == m1, lane, E), axis=1, keepdims=True)
    masked = jnp.where(lane == i1, -jnp.inf, logits)
    m2 = jnp.max(masked, axis=1, keepdims=True)
    i2 = jnp.min(jnp.where(masked == m2, lane, E), axis=1, keepdims=True)
    w1 = jax.nn.sigmoid(m1 - m2)
    topi_ref[...] = jnp.concatenate([i1, i2], axis=1).astype(jnp.int32)
    topw_ref[...] = jnp.concatenate([w1, 1.0 - w1], axis=1)


_router = pl.pallas_call(
    _router_body,
    grid=(T // _TB,),
    in_specs=[
        pl.BlockSpec((_TB, D), lambda t: (t, 0)),
        pl.BlockSpec((D, E), lambda t: (0, 0)),
    ],
    out_specs=[
        pl.BlockSpec((_TB, TOPK), lambda t: (t, 0)),
        pl.BlockSpec((_TB, TOPK), lambda t: (t, 0)),
    ],
    out_shape=[
        jax.ShapeDtypeStruct((T, TOPK), jnp.int32),
        jax.ShapeDtypeStruct((T, TOPK), jnp.float32),
    ],
)


def _dispatch(topi, topw):
    eflat = topi.reshape(-1)
    onehot = (eflat[:, None] == jnp.arange(E, dtype=jnp.int32)[None, :]).astype(jnp.int32)
    pos = jnp.cumsum(onehot, axis=0)
    counts = pos[-1]
    rank = jnp.sum(onehot * pos, axis=1) - 1
    nblk = (counts + B - 1) // B
    blk_cum = jnp.cumsum(nblk)
    nb_active = blk_cum[-1]
    bb = jnp.arange(NB, dtype=jnp.int32)
    blk_expert = jnp.sum((bb[:, None] >= blk_cum[None, :]).astype(jnp.int32), axis=1)
    blk_expert = jnp.minimum(blk_expert, E - 1)
    sp = jnp.concatenate([blk_expert, nb_active[None]]).astype(jnp.int32)
    pad_off = (blk_cum - nblk) * B
    ps = pad_off[eflat] + rank
    w_pad = jnp.zeros((NP,), jnp.float32).at[ps].set(topw.reshape(-1))
    cpos = ps.reshape(T, TOPK)
    ps2 = ps.reshape(_NW, T // _NW, TOPK).transpose(0, 2, 1)
    return sp, ps2, w_pad, cpos


@functools.cache
def _sc_dispatch():
    @functools.partial(
        pl.kernel,
        out_type=jax.ShapeDtypeStruct((NP, D), jnp.float32),
        mesh=_sc_mesh(),
        scratch_types=[
            pltpu.VMEM((TOPK, T // _NW), jnp.int32),
            pltpu.VMEM((T // _NW, D), jnp.float32),
            pltpu.SemaphoreType.DMA,
            pltpu.SemaphoreType.DMA,
        ],
    )
    def dispatch(x_hbm, ps2_hbm, xs_hbm, idx_v, rows_v, s0, s1):
        wid = lax.axis_index("s") * 2 + lax.axis_index("c")
        tw = T // _NW
        pltpu.sync_copy(x_hbm.at[pl.ds(wid * tw, tw)], rows_v)
        pltpu.sync_copy(ps2_hbm.at[wid], idx_v)
        cp0 = pltpu.async_copy(rows_v, xs_hbm.at[idx_v.at[0]], s0)
        cp1 = pltpu.async_copy(rows_v, xs_hbm.at[idx_v.at[1]], s1)
        cp0.wait()
        cp1.wait()

    return dispatch


def _ffn_body(sp_ref, xs_ref, w_ref, wg_ref, wu_ref, wd_ref, ys_ref):
    b = pl.program_id(0)
    j = pl.program_id(1)
    nact = sp_ref[NB]

    @pl.when(b < nact)
    def _():
        x = xs_ref[...]
        g = jnp.dot(x, wg_ref[0], preferred_element_type=jnp.float32)
        u = jnp.dot(x, wu_ref[0], preferred_element_type=jnp.float32)
        h = g * jax.nn.sigmoid(g) * u
        part = jnp.dot(h, wd_ref[0], preferred_element_type=jnp.float32) * w_ref[...]

        @pl.when(j == 0)
        def _():
            ys_ref[...] = part

        @pl.when(j > 0)
        def _():
            ys_ref[...] += part


_ffn = pl.pallas_call(
    _ffn_body,
    grid_spec=pltpu.PrefetchScalarGridSpec(
        num_scalar_prefetch=1,
        grid=(NB, NJ),
        in_specs=[
            pl.BlockSpec(
                (B, D), lambda b, j, sp: (jnp.minimum(b, sp[NB] - 1), 0)),
            pl.BlockSpec(
                (B, 1), lambda b, j, sp: (jnp.minimum(b, sp[NB] - 1), 0)),
            pl.BlockSpec(
                (1, D, DBLK),
                lambda b, j, sp: (sp[jnp.minimum(b, sp[NB] - 1)], 0, j)),
            pl.BlockSpec(
                (1, D, DBLK),
                lambda b, j, sp: (sp[jnp.minimum(b, sp[NB] - 1)], 0, j)),
            pl.BlockSpec(
                (1, DBLK, D),
                lambda b, j, sp: (sp[jnp.minimum(b, sp[NB] - 1)], j, 0)),
        ],
        out_specs=pl.BlockSpec(
            (B, D), lambda b, j, sp: (jnp.minimum(b, sp[NB] - 1), 0)),
    ),
    out_shape=jax.ShapeDtypeStruct((NP, D), jnp.float32),
    compiler_params=pltpu.CompilerParams(
        dimension_semantics=("arbitrary", "arbitrary")),
)


@functools.cache
def _sc_combine():
    @functools.partial(
        pl.kernel,
        out_type=jax.ShapeDtypeStruct((T, D), jnp.float32),
        mesh=_sc_mesh(),
        scratch_types=[
            pltpu.VMEM((16,), jnp.int32),
            pltpu.VMEM((16,), jnp.int32),
            pltpu.VMEM((16, D), jnp.float32),
            pltpu.VMEM((16, D), jnp.float32),
            pltpu.SemaphoreType.DMA,
            pltpu.SemaphoreType.DMA,
        ],
    )
    def combine(ys_hbm, c0_hbm, c1_hbm, out_hbm,
                i0_v, i1_v, a_v, b_v, sem0, sem1):
        wid = lax.axis_index("s") * 2 + lax.axis_index("c")
        for r in range(_TW // 16):
            base = wid * _TW + r * 16
            pltpu.sync_copy(c0_hbm.at[pl.ds(base, 16)], i0_v)
            pltpu.sync_copy(c1_hbm.at[pl.ds(base, 16)], i1_v)
            cp0 = pltpu.async_copy(ys_hbm.at[i0_v], a_v, sem0)
            cp1 = pltpu.async_copy(ys_hbm.at[i1_v], b_v, sem1)
            cp0.wait()
            cp1.wait()

            def body(i, carry):
                l = i // (D // 16)
                sl = pl.ds((i % (D // 16)) * 16, 16)
                a_v[l, sl] = a_v[l, sl] + b_v[l, sl]
                return carry

            lax.fori_loop(0, 16 * (D // 16), body, 0, unroll=4)
            pltpu.sync_copy(a_v, out_hbm.at[pl.ds(base, 16)])

    return combine


def kernel(hidden_states, Wg, W_gate, W_up, W_down):
    x = hidden_states
    topi, topw = _router(x, Wg)
    sp, ps2, w_pad, cpos = _dispatch(topi, topw)
    xs = _sc_dispatch()(x, ps2)
    ys = _ffn(sp, xs, w_pad[:, None], W_gate, W_up, W_down)
    return _sc_combine()(ys, cpos[:, 0], cpos[:, 1])
